# Initial kernel scaffold; baseline (speedup 1.0000x reference)
import jax
import jax.numpy as jnp
from jax.experimental import pallas as pl


def kernel(x, table):
    raise NotImplementedError("write your pallas kernel here")


# speedup vs baseline: 1.2660x; 1.2660x over previous
import functools
import jax
import jax.numpy as jnp
from jax import lax
from jax.experimental import pallas as pl
from jax.experimental.pallas import tpu as pltpu
from jax.experimental.pallas import tpu_sc as plsc

B, L, D = 4096, 200, 128
NC, NS = 2, 16
NW = NC * NS
BPW = B // NW
C0, C1 = 104, 96


def _body(x_hbm, tab_hbm, out_hbm, idx_v, rows_v, tr_v, gsem):
    wid = lax.axis_index("s") * NC + lax.axis_index("c")
    lane = lax.iota(jnp.int32, 16)

    def per_row(i, carry):
        b = wid * BPW + i
        xoff = pl.multiple_of(b * L, 8)
        pltpu.sync_copy(x_hbm.at[pl.ds(xoff, C0)], idx_v.at[0, pl.ds(0, C0)])
        pltpu.sync_copy(x_hbm.at[pl.ds(xoff + C0, C1)], idx_v.at[1, pl.ds(0, C1)])
        cp0 = pltpu.async_copy(
            tab_hbm.at[idx_v.at[0, pl.ds(0, C0)]], rows_v.at[pl.ds(0, C0)], gsem)
        cp1 = pltpu.async_copy(
            tab_hbm.at[idx_v.at[1, pl.ds(0, C1)]], rows_v.at[pl.ds(C0, C1)], gsem)
        cp0.wait()
        cp1.wait()

        def tr(l, c):
            base = lane * L + l
            for j in range(D // 16):
                v = rows_v[l, pl.ds(j * 16, 16)]
                plsc.store_scatter(tr_v, [base + j * 16 * L], v)
            return c

        lax.fori_loop(0, L, tr, 0)
        ooff = pl.multiple_of(b * (D * L), 8)
        pltpu.sync_copy(tr_v, out_hbm.at[pl.ds(ooff, D * L)])
        return carry

    lax.fori_loop(0, BPW, per_row, 0)


def kernel(x, table):
    x = x.astype(jnp.int32).reshape(B * L)
    mesh = plsc.VectorSubcoreMesh(core_axis_name="c", subcore_axis_name="s")
    out = pl.kernel(
        _body,
        mesh=mesh,
        out_type=jax.ShapeDtypeStruct((B * D * L,), jnp.float32),
        compiler_params=pltpu.CompilerParams(needs_layout_passes=False),
        scratch_types=[
            pltpu.VMEM((2, 128), jnp.int32),
            pltpu.VMEM((L, D), jnp.float32),
            pltpu.VMEM((D * L,), jnp.float32),
            pltpu.SemaphoreType.DMA,
        ],
    )(x, table)
    return out.reshape(B, D, L)


# speedup vs baseline: 1.5848x; 1.2518x over previous
import functools
import jax
import jax.numpy as jnp
from jax import lax
from jax.experimental import pallas as pl
from jax.experimental.pallas import tpu as pltpu
from jax.experimental.pallas import tpu_sc as plsc

B, L, D = 4096, 200, 128
NC, NS = 2, 16
NW = NC * NS
BPW = B // NW
C0, C1 = 104, 96


def _body(x_hbm, tab_hbm, out_hbm, idx_v, rows0, rows1, tr0, tr1,
          g0, g1, o0, o1):
    wid = lax.axis_index("s") * NC + lax.axis_index("c")
    lane = lax.iota(jnp.int32, 16)
    row0 = wid * BPW

    xoff = pl.multiple_of(row0 * L, 8)
    pltpu.sync_copy(x_hbm.at[pl.ds(xoff, BPW * L)], idx_v)

    def fire_gather(i, dst, sem):
        off = pl.multiple_of(i * L, 8)
        pltpu.async_copy(tab_hbm.at[idx_v.at[pl.ds(off, C0)]],
                         dst.at[pl.ds(0, C0)], sem)
        pltpu.async_copy(tab_hbm.at[idx_v.at[pl.ds(off + C0, C1)]],
                         dst.at[pl.ds(C0, C1)], sem)

    def wait_gather(dst, sem):
        pltpu.make_async_copy(tab_hbm.at[idx_v.at[pl.ds(0, C0)]],
                              dst.at[pl.ds(0, C0)], sem).wait()
        pltpu.make_async_copy(tab_hbm.at[idx_v.at[pl.ds(0, C1)]],
                              dst.at[pl.ds(C0, C1)], sem).wait()

    def transpose(rows_s, tr_s):
        def tr(l, c):
            base = lane * L + l
            for j in range(D // 16):
                v = rows_s[l, pl.ds(j * 16, 16)]
                plsc.store_scatter(tr_s, [base + j * 16 * L], v)
            return c
        lax.fori_loop(0, L, tr, 0)

    def out_slice(i):
        ooff = pl.multiple_of((row0 + i) * (D * L), 8)
        return out_hbm.at[pl.ds(ooff, D * L)]

    fire_gather(row0 - row0 + 0, rows0, g0)
    fire_gather(1, rows1, g1)

    def step(p, carry):
        for s, rows_s, tr_s, gs, os in ((0, rows0, tr0, g0, o0),
                                        (1, rows1, tr1, g1, o1)):
            i = 2 * p + s
            wait_gather(rows_s, gs)

            @pl.when(p > 0)
            def _():
                pltpu.make_async_copy(tr_s, out_slice(0), os).wait()

            transpose(rows_s, tr_s)

            @pl.when(p < BPW // 2 - 1)
            def _():
                fire_gather(i + 2, rows_s, gs)

            pltpu.async_copy(tr_s, out_slice(i), os)
        return carry

    lax.fori_loop(0, BPW // 2, step, 0)
    pltpu.make_async_copy(tr0, out_slice(0), o0).wait()
    pltpu.make_async_copy(tr1, out_slice(0), o1).wait()


def kernel(x, table):
    x = x.astype(jnp.int32).reshape(B * L)
    mesh = plsc.VectorSubcoreMesh(core_axis_name="c", subcore_axis_name="s")
    out = pl.kernel(
        _body,
        mesh=mesh,
        out_type=jax.ShapeDtypeStruct((B * D * L,), jnp.float32),
        compiler_params=pltpu.CompilerParams(needs_layout_passes=False),
        scratch_types=[
            pltpu.VMEM((BPW * L,), jnp.int32),
            pltpu.VMEM((L, D), jnp.float32),
            pltpu.VMEM((L, D), jnp.float32),
            pltpu.VMEM((D * L,), jnp.float32),
            pltpu.VMEM((D * L,), jnp.float32),
            pltpu.SemaphoreType.DMA,
            pltpu.SemaphoreType.DMA,
            pltpu.SemaphoreType.DMA,
            pltpu.SemaphoreType.DMA,
        ],
    )(x, table)
    return out.reshape(B, D, L)
